# baseline (speedup 1.0000x reference)
import functools

import jax
import jax.numpy as jnp
from jax.experimental import pallas as pl
from jax.experimental.pallas import tpu as pltpu

_U = 8


def _round_up(a: int, b: int) -> int:
    return (a + b - 1) // b * b


def _fused_kernel(E_ref, idx_ref, rcat_ref, out_ref,
                  g_ref, t_ref, s_sm, o_sm, sems, *, dim, rel_count,
                  tile_m, nblk):
    i = pl.program_id(0)

    cp_s = pltpu.make_async_copy(idx_ref.at[i], s_sm, sems.at[0])
    cp_o = pltpu.make_async_copy(idx_ref.at[nblk + i], o_sm, sems.at[1])
    cp_s.start()
    cp_o.start()

    ridx = idx_ref[2 * nblk + i]
    rel_iota = jax.lax.broadcasted_iota(jnp.int32, (rel_count, tile_m), 0)
    onehot_t = (rel_iota == ridx).astype(jnp.bfloat16)
    g = jax.lax.dot_general(
        onehot_t, rcat_ref[...],
        dimension_numbers=(((0,), (0,)), ((), ())),
        preferred_element_type=jnp.float32)
    g_ref[...] = g.astype(jnp.bfloat16)

    cp_s.wait()
    cp_o.wait()

    for c in range(tile_m // _U):
        base = c * _U
        srows = []
        orows = []
        for u in range(_U):
            srows.append(E_ref[s_sm[0, base + u], 0])
            orows.append(E_ref[o_sm[0, base + u], 0])
        s8 = jnp.stack(srows, axis=0)
        o8 = jnp.stack(orows, axis=0)
        g8 = g_ref[pl.ds(base, _U), :]
        t8 = s8 * g8[:, :dim].astype(jnp.float32) \
            + o8 * g8[:, dim:].astype(jnp.float32)
        t_ref[pl.ds(base, _U), :] = t8.astype(jnp.bfloat16)

    out_ref[...] = jnp.sum(t_ref[...], axis=-1, keepdims=True,
                           dtype=jnp.float32)


@jax.jit
def kernel(E, R_head, R_tail, s_idx, r_idx, o_idx):
    batch, x = s_idx.shape
    ec, dim = E.shape
    rel_count = R_head.shape[0]
    n = batch * x

    tile_m = 2048
    rows = _round_up(n, tile_m)
    nblk = rows // tile_m

    def _pad_flat(idx):
        flat = idx.reshape(-1).astype(jnp.int32)
        return jnp.pad(flat, (0, rows - n))

    idx_cat = jnp.concatenate(
        [_pad_flat(s_idx), _pad_flat(o_idx), _pad_flat(r_idx)]
    ).reshape(3 * nblk, 1, tile_m)
    rcat = jnp.concatenate([R_head, R_tail], axis=-1).astype(jnp.bfloat16)
    E3 = E.reshape(ec, 1, dim)

    scores = pl.pallas_call(
        functools.partial(_fused_kernel, dim=dim, rel_count=rel_count,
                          tile_m=tile_m, nblk=nblk),
        out_shape=jax.ShapeDtypeStruct((rows, 1), jnp.float32),
        grid=(nblk,),
        in_specs=[
            pl.BlockSpec((ec, 1, dim), lambda i: (0, 0, 0)),
            pl.BlockSpec((3 * nblk, 1, tile_m), lambda i: (0, 0, 0)),
            pl.BlockSpec((rel_count, 2 * dim), lambda i: (0, 0)),
        ],
        out_specs=pl.BlockSpec((tile_m, 1), lambda i: (i, 0)),
        scratch_shapes=[
            pltpu.VMEM((tile_m, 2 * dim), jnp.bfloat16),
            pltpu.VMEM((tile_m, dim), jnp.bfloat16),
            pltpu.SMEM((1, tile_m), jnp.int32),
            pltpu.SMEM((1, tile_m), jnp.int32),
            pltpu.SemaphoreType.DMA((2,)),
        ],
        compiler_params=pltpu.CompilerParams(
            dimension_semantics=("parallel",),
            vmem_limit_bytes=63 * 1024 * 1024,
        ),
    )(E3, idx_cat, rcat)

    return scores.reshape(rows)[:n].reshape(batch, x)


# speedup vs baseline: 1.0920x; 1.0920x over previous
import functools

import jax
import jax.numpy as jnp
from jax.experimental import pallas as pl
from jax.experimental.pallas import tpu as pltpu

_U = 8


def _round_up(a: int, b: int) -> int:
    return (a + b - 1) // b * b


def _fused_kernel(E_ref, sidx_ref, oidx_ref, ridx_ref, rh_ref, rt_ref,
                  out_ref, g_ref, st_ref, ot_ref, rcat_ref, s_sm, o_sm, sems,
                  *, dim, rel_count, tile_m, rows_per_blk, x):
    i = pl.program_id(0)

    @pl.when(i == 0)
    def _init():
        rcat_ref[:, :dim] = rh_ref[...].astype(jnp.bfloat16)
        rcat_ref[:, dim:] = rt_ref[...].astype(jnp.bfloat16)

    cp_s = pltpu.make_async_copy(sidx_ref, s_sm, sems.at[0])
    cp_o = pltpu.make_async_copy(oidx_ref, o_sm, sems.at[1])
    cp_s.start()
    cp_o.start()

    ridx = ridx_ref[...].reshape(1, tile_m)
    rel_iota = jax.lax.broadcasted_iota(jnp.int32, (rel_count, tile_m), 0)
    onehot_t = (rel_iota == ridx).astype(jnp.bfloat16)
    g_ref[...] = jax.lax.dot_general(
        onehot_t, rcat_ref[...],
        dimension_numbers=(((0,), (0,)), ((), ())),
        preferred_element_type=jnp.float32)

    cp_s.wait()
    cp_o.wait()

    for c in range(tile_m // _U):
        base = c * _U
        srows = []
        orows = []
        for u in range(_U):
            m = base + u
            srows.append(E_ref[s_sm[m // x, m % x], 0])
            orows.append(E_ref[o_sm[m // x, m % x], 0])
        st_ref[pl.ds(base, _U), :] = jnp.stack(srows, axis=0)
        ot_ref[pl.ds(base, _U), :] = jnp.stack(orows, axis=0)

    s = st_ref[...]
    o = ot_ref[...]
    g = g_ref[...]
    out_ref[...] = jnp.sum(s * g[:, :dim] + o * g[:, dim:],
                           axis=-1, keepdims=True)


@jax.jit
def kernel(E, R_head, R_tail, s_idx, r_idx, o_idx):
    batch, x = s_idx.shape
    ec, dim = E.shape
    rel_count = R_head.shape[0]
    n = batch * x

    tile_m = 2048
    rows_per_blk = tile_m // x
    nblk = n // tile_m
    E3 = E.reshape(ec, 1, dim)

    scores = pl.pallas_call(
        functools.partial(_fused_kernel, dim=dim, rel_count=rel_count,
                          tile_m=tile_m, rows_per_blk=rows_per_blk, x=x),
        out_shape=jax.ShapeDtypeStruct((n, 1), jnp.float32),
        grid=(nblk,),
        in_specs=[
            pl.BlockSpec((ec, 1, dim), lambda i: (0, 0, 0)),
            pl.BlockSpec((rows_per_blk, x), lambda i: (i, 0)),
            pl.BlockSpec((rows_per_blk, x), lambda i: (i, 0)),
            pl.BlockSpec((rows_per_blk, x), lambda i: (i, 0)),
            pl.BlockSpec((rel_count, dim), lambda i: (0, 0)),
            pl.BlockSpec((rel_count, dim), lambda i: (0, 0)),
        ],
        out_specs=pl.BlockSpec((tile_m, 1), lambda i: (i, 0)),
        scratch_shapes=[
            pltpu.VMEM((tile_m, 2 * dim), jnp.float32),
            pltpu.VMEM((tile_m, dim), jnp.float32),
            pltpu.VMEM((tile_m, dim), jnp.float32),
            pltpu.VMEM((rel_count, 2 * dim), jnp.bfloat16),
            pltpu.SMEM((rows_per_blk, x), jnp.int32),
            pltpu.SMEM((rows_per_blk, x), jnp.int32),
            pltpu.SemaphoreType.DMA((2,)),
        ],
        compiler_params=pltpu.CompilerParams(
            dimension_semantics=("arbitrary",),
            vmem_limit_bytes=63 * 1024 * 1024,
        ),
    )(E3, s_idx, o_idx, r_idx, R_head, R_tail)

    return scores.reshape(batch, x)


# speedup vs baseline: 1.1259x; 1.0310x over previous
import functools

import jax
import jax.numpy as jnp
from jax.experimental import pallas as pl
from jax.experimental.pallas import tpu as pltpu

_U = 8


def _round_up(a: int, b: int) -> int:
    return (a + b - 1) // b * b


def _fused_kernel(E_ref, sidx_ref, oidx_ref, ridx_ref, rh_ref, rt_ref,
                  out_ref, g_ref, st_ref, ot_ref, rcat_ref, s_sm, o_sm, sems,
                  *, dim, rel_count, tile_m, rows_per_blk, x):
    i = pl.program_id(0)

    @pl.when(i == 0)
    def _init():
        rcat_ref[:, :dim] = rh_ref[...].astype(jnp.bfloat16)
        rcat_ref[:, dim:] = rt_ref[...].astype(jnp.bfloat16)

    cp_s = pltpu.make_async_copy(sidx_ref, s_sm, sems.at[0])
    cp_o = pltpu.make_async_copy(oidx_ref, o_sm, sems.at[1])
    cp_s.start()
    cp_o.start()

    ridx = ridx_ref[...].reshape(1, tile_m)
    rel_iota = jax.lax.broadcasted_iota(jnp.int32, (rel_count, tile_m), 0)
    onehot_t = (rel_iota == ridx).astype(jnp.bfloat16)
    g_ref[...] = jax.lax.dot_general(
        onehot_t, rcat_ref[...],
        dimension_numbers=(((0,), (0,)), ((), ())),
        preferred_element_type=jnp.float32)

    cp_s.wait()
    cp_o.wait()

    for c in range(tile_m // _U):
        base = c * _U
        srows = []
        orows = []
        for u in range(_U):
            m = base + u
            srows.append(E_ref[s_sm[m // x, m % x], 0])
            orows.append(E_ref[o_sm[m // x, m % x], 0])
        st_ref[pl.ds(base, _U), :] = jnp.stack(srows, axis=0)
        ot_ref[pl.ds(base, _U), :] = jnp.stack(orows, axis=0)

    s = st_ref[...]
    o = ot_ref[...]
    g = g_ref[...]
    t = (s * g[:, :dim] + o * g[:, dim:]).astype(jnp.bfloat16)
    ones_row = jnp.ones((1, dim), jnp.bfloat16)
    score_row = jax.lax.dot_general(
        ones_row, t,
        dimension_numbers=(((1,), (1,)), ((), ())),
        preferred_element_type=jnp.float32)
    out_ref[...] = score_row.reshape(rows_per_blk, x)


@jax.jit
def kernel(E, R_head, R_tail, s_idx, r_idx, o_idx):
    batch, x = s_idx.shape
    ec, dim = E.shape
    rel_count = R_head.shape[0]
    n = batch * x

    tile_m = 2048
    rows_per_blk = tile_m // x
    nblk = n // tile_m
    E3 = E.reshape(ec, 1, dim)

    scores = pl.pallas_call(
        functools.partial(_fused_kernel, dim=dim, rel_count=rel_count,
                          tile_m=tile_m, rows_per_blk=rows_per_blk, x=x),
        out_shape=jax.ShapeDtypeStruct((batch, x), jnp.float32),
        grid=(nblk,),
        in_specs=[
            pl.BlockSpec((ec, 1, dim), lambda i: (0, 0, 0)),
            pl.BlockSpec((rows_per_blk, x), lambda i: (i, 0)),
            pl.BlockSpec((rows_per_blk, x), lambda i: (i, 0)),
            pl.BlockSpec((rows_per_blk, x), lambda i: (i, 0)),
            pl.BlockSpec((rel_count, dim), lambda i: (0, 0)),
            pl.BlockSpec((rel_count, dim), lambda i: (0, 0)),
        ],
        out_specs=pl.BlockSpec((rows_per_blk, x), lambda i: (i, 0)),
        scratch_shapes=[
            pltpu.VMEM((tile_m, 2 * dim), jnp.float32),
            pltpu.VMEM((tile_m, dim), jnp.float32),
            pltpu.VMEM((tile_m, dim), jnp.float32),
            pltpu.VMEM((rel_count, 2 * dim), jnp.bfloat16),
            pltpu.SMEM((rows_per_blk, x), jnp.int32),
            pltpu.SMEM((rows_per_blk, x), jnp.int32),
            pltpu.SemaphoreType.DMA((2,)),
        ],
        compiler_params=pltpu.CompilerParams(
            dimension_semantics=("arbitrary",),
            vmem_limit_bytes=63 * 1024 * 1024,
        ),
    )(E3, s_idx, o_idx, r_idx, R_head, R_tail)

    return scores


# speedup vs baseline: 1.1796x; 1.0477x over previous
import functools

import jax
import jax.numpy as jnp
from jax.experimental import pallas as pl
from jax.experimental.pallas import tpu as pltpu

_U = 8


def _round_up(a: int, b: int) -> int:
    return (a + b - 1) // b * b


def _fused_kernel(E_ref, sidx_ref, oidx_ref, ridx_ref, rh_ref, rt_ref,
                  out_ref, g_ref, st_ref, ot_ref, rcat_ref, s_sm, o_sm, sems,
                  *, dim, rel_count, tile_m, rows_per_blk, x):
    i = pl.program_id(0)

    @pl.when(i == 0)
    def _init():
        rcat_ref[:, :dim] = rh_ref[...].astype(jnp.bfloat16)
        rcat_ref[:, dim:] = rt_ref[...].astype(jnp.bfloat16)

    cp_s = pltpu.make_async_copy(sidx_ref, s_sm, sems.at[0])
    cp_o = pltpu.make_async_copy(oidx_ref, o_sm, sems.at[1])
    cp_s.start()
    cp_o.start()

    ridx = ridx_ref[...].reshape(1, tile_m)
    half = tile_m // 2
    rel_iota = jax.lax.broadcasted_iota(jnp.int32, (rel_count, half), 0)
    for h in range(2):
        oh = (rel_iota == ridx[:, h * half:(h + 1) * half]).astype(
            jnp.bfloat16)
        g_ref[pl.ds(h * half, half), :] = jax.lax.dot_general(
            oh, rcat_ref[...],
            dimension_numbers=(((0,), (0,)), ((), ())),
            preferred_element_type=jnp.float32)

    cp_s.wait()
    cp_o.wait()

    for c in range(tile_m // _U):
        base = c * _U
        srows = []
        orows = []
        for u in range(_U):
            m = base + u
            srows.append(E_ref[s_sm[m // x, m % x], 0])
            orows.append(E_ref[o_sm[m // x, m % x], 0])
        st_ref[pl.ds(base, _U), :] = jnp.stack(srows, axis=0)
        ot_ref[pl.ds(base, _U), :] = jnp.stack(orows, axis=0)

    s = st_ref[...]
    o = ot_ref[...]
    g = g_ref[...]
    t = (s * g[:, :dim] + o * g[:, dim:]).astype(jnp.bfloat16)
    ones_row = jnp.ones((1, dim), jnp.bfloat16)
    score_row = jax.lax.dot_general(
        ones_row, t,
        dimension_numbers=(((1,), (1,)), ((), ())),
        preferred_element_type=jnp.float32)
    out_ref[...] = score_row.reshape(rows_per_blk, x)


@jax.jit
def kernel(E, R_head, R_tail, s_idx, r_idx, o_idx):
    batch, x = s_idx.shape
    ec, dim = E.shape
    rel_count = R_head.shape[0]
    n = batch * x

    tile_m = 4096
    rows_per_blk = tile_m // x
    nblk = n // tile_m
    E3 = E.reshape(ec, 1, dim)

    scores = pl.pallas_call(
        functools.partial(_fused_kernel, dim=dim, rel_count=rel_count,
                          tile_m=tile_m, rows_per_blk=rows_per_blk, x=x),
        out_shape=jax.ShapeDtypeStruct((batch, x), jnp.float32),
        grid=(nblk,),
        in_specs=[
            pl.BlockSpec((ec, 1, dim), lambda i: (0, 0, 0)),
            pl.BlockSpec((rows_per_blk, x), lambda i: (i, 0)),
            pl.BlockSpec((rows_per_blk, x), lambda i: (i, 0)),
            pl.BlockSpec((rows_per_blk, x), lambda i: (i, 0)),
            pl.BlockSpec((rel_count, dim), lambda i: (0, 0)),
            pl.BlockSpec((rel_count, dim), lambda i: (0, 0)),
        ],
        out_specs=pl.BlockSpec((rows_per_blk, x), lambda i: (i, 0)),
        scratch_shapes=[
            pltpu.VMEM((tile_m, 2 * dim), jnp.float32),
            pltpu.VMEM((tile_m, dim), jnp.float32),
            pltpu.VMEM((tile_m, dim), jnp.float32),
            pltpu.VMEM((rel_count, 2 * dim), jnp.bfloat16),
            pltpu.SMEM((rows_per_blk, x), jnp.int32),
            pltpu.SMEM((rows_per_blk, x), jnp.int32),
            pltpu.SemaphoreType.DMA((2,)),
        ],
        compiler_params=pltpu.CompilerParams(
            dimension_semantics=("arbitrary",),
            vmem_limit_bytes=63 * 1024 * 1024,
        ),
    )(E3, s_idx, o_idx, r_idx, R_head, R_tail)

    return scores


# speedup vs baseline: 1.1799x; 1.0002x over previous
import functools

import jax
import jax.numpy as jnp
from jax.experimental import pallas as pl
from jax.experimental.pallas import tpu as pltpu

_U = 8


def _fused_kernel(E_ref, sidx_ref, oidx_ref, ridx_ref, rh_ref, rt_ref,
                  out_ref, g_ref, st_ref, ot_ref, rcat_ref, s_sm, o_sm, sems,
                  *, dim, rel_count, tile_m, rows_per_blk, x):
    i = pl.program_id(0)

    @pl.when(i == 0)
    def _init():
        rcat_ref[:, :dim] = rh_ref[...].astype(jnp.bfloat16)
        rcat_ref[:, dim:] = rt_ref[...].astype(jnp.bfloat16)

    cp_s = pltpu.make_async_copy(sidx_ref, s_sm, sems.at[0])
    cp_o = pltpu.make_async_copy(oidx_ref, o_sm, sems.at[1])
    cp_s.start()
    cp_o.start()

    ridx = ridx_ref[...].reshape(1, tile_m)
    half = tile_m // 2
    rel_iota = jax.lax.broadcasted_iota(jnp.int32, (rel_count, half), 0)
    for h in range(2):
        oh = (rel_iota == ridx[:, h * half:(h + 1) * half]).astype(
            jnp.bfloat16)
        g_ref[pl.ds(h * half, half), :] = jax.lax.dot_general(
            oh, rcat_ref[...],
            dimension_numbers=(((0,), (0,)), ((), ())),
            preferred_element_type=jnp.float32)

    cp_s.wait()
    cp_o.wait()

    for c in range(tile_m // _U):
        base = c * _U
        srows = []
        orows = []
        for u in range(_U):
            m = base + u
            srows.append(E_ref[s_sm[m // x, m % x], 0])
            orows.append(E_ref[o_sm[m // x, m % x], 0])
        st_ref[pl.ds(base, _U), :] = jnp.stack(srows, axis=0)
        ot_ref[pl.ds(base, _U), :] = jnp.stack(orows, axis=0)

    s = st_ref[...]
    o = ot_ref[...]
    g = g_ref[...]
    t = (s * g[:, :dim] + o * g[:, dim:]).astype(jnp.bfloat16)
    ones_row = jnp.ones((1, dim), jnp.bfloat16)
    score_row = jax.lax.dot_general(
        ones_row, t,
        dimension_numbers=(((1,), (1,)), ((), ())),
        preferred_element_type=jnp.float32)
    out_ref[...] = score_row.reshape(rows_per_blk, x)


@jax.jit
def kernel(E, R_head, R_tail, s_idx, r_idx, o_idx):
    batch, x = s_idx.shape
    ec, dim = E.shape
    rel_count = R_head.shape[0]
    n = batch * x

    tile_m = 4096
    while tile_m > x and (n % tile_m or tile_m % x):
        tile_m //= 2
    rows_per_blk = tile_m // x
    nblk = n // tile_m
    E3 = E.reshape(ec, 1, dim)

    scores = pl.pallas_call(
        functools.partial(_fused_kernel, dim=dim, rel_count=rel_count,
                          tile_m=tile_m, rows_per_blk=rows_per_blk, x=x),
        out_shape=jax.ShapeDtypeStruct((batch, x), jnp.float32),
        grid=(nblk,),
        in_specs=[
            pl.BlockSpec((ec, 1, dim), lambda i: (0, 0, 0)),
            pl.BlockSpec((rows_per_blk, x), lambda i: (i, 0)),
            pl.BlockSpec((rows_per_blk, x), lambda i: (i, 0)),
            pl.BlockSpec((rows_per_blk, x), lambda i: (i, 0)),
            pl.BlockSpec((rel_count, dim), lambda i: (0, 0)),
            pl.BlockSpec((rel_count, dim), lambda i: (0, 0)),
        ],
        out_specs=pl.BlockSpec((rows_per_blk, x), lambda i: (i, 0)),
        scratch_shapes=[
            pltpu.VMEM((tile_m, 2 * dim), jnp.float32),
            pltpu.VMEM((tile_m, dim), jnp.float32),
            pltpu.VMEM((tile_m, dim), jnp.float32),
            pltpu.VMEM((rel_count, 2 * dim), jnp.bfloat16),
            pltpu.SMEM((rows_per_blk, x), jnp.int32),
            pltpu.SMEM((rows_per_blk, x), jnp.int32),
            pltpu.SemaphoreType.DMA((2,)),
        ],
        compiler_params=pltpu.CompilerParams(
            dimension_semantics=("arbitrary",),
            vmem_limit_bytes=63 * 1024 * 1024,
        ),
    )(E3, s_idx, o_idx, r_idx, R_head, R_tail)

    return scores
